# baseline (speedup 1.0000x reference)
import functools

import jax
import jax.numpy as jnp
from jax import lax
from jax.experimental import pallas as pl
from jax.experimental.pallas import tpu as pltpu
from jax.experimental.pallas import tpu_sc as plsc

ROWS = 1000
COLS = 65536
L = 16

NW = 32
RPW = 32
SC_ROWS = NW * RPW
RG = 4
NG = RPW // RG
C = 4096
NCH = COLS // C
U = 4

TC_START = 240
BLOCK_R = 40
TC_ROWS = ROWS - TC_START


def _sc_body(syn_hbm, img_hbm, out_hbm, feat, buf0, buf1, res, sem0, sem1):
    wid = lax.axis_index("s") * 2 + lax.axis_index("c")
    base = wid * RPW

    img_cp = pltpu.async_copy(img_hbm, feat, sem1)
    s0 = jnp.minimum(base, ROWS - RG)
    pltpu.async_copy(syn_hbm.at[pl.ds(s0, RG), pl.ds(0, C)], buf0, sem0)
    img_cp.wait()

    ones = jnp.full((L,), 1.0, jnp.float32)
    zeros = jnp.zeros((L,), jnp.float32)

    def _thresh(i, _):
        for u in range(8):
            off = (i * 8 + u) * L
            v = feat[pl.ds(off, L)]
            feat[pl.ds(off, L)] = jnp.where(v > 0.7, ones, zeros)
        return 0

    lax.fori_loop(0, COLS // (L * 8), _thresh, 0)

    def _inner(bufc, k, accs):
        def body(j, accs):
            for u in range(U):
                off = (j * U + u) * L
                fv = feat[pl.ds(k * C + off, L)]
                accs = tuple(
                    accs[r] + bufc[r, pl.ds(off, L)] * fv for r in range(RG)
                )
            return accs

        return lax.fori_loop(0, 1, body, accs)

    for g in range(NG):
        sg = jnp.minimum(base + g * RG, ROWS - RG)
        sg_next = jnp.minimum(base + (g + 1) * RG, ROWS - RG)
        accs = tuple(jnp.zeros((L,), jnp.float32) for _ in range(RG))

        def _pair(p, accs, sg=sg, sg_next=sg_next):
            k0 = 2 * p
            pltpu.make_async_copy(
                syn_hbm.at[pl.ds(sg, RG), pl.ds(0, C)], buf0, sem0
            ).wait()
            pltpu.async_copy(
                syn_hbm.at[pl.ds(sg, RG), pl.ds((k0 + 1) * C, C)], buf1, sem1
            )
            accs = _inner(buf0, k0, accs)
            pltpu.make_async_copy(
                syn_hbm.at[pl.ds(sg, RG), pl.ds(0, C)], buf1, sem1
            ).wait()

            @pl.when(k0 + 2 < NCH)
            def _():
                pltpu.async_copy(
                    syn_hbm.at[pl.ds(sg, RG), pl.ds((k0 + 2) * C, C)], buf0, sem0
                )

            @pl.when(k0 + 2 >= NCH)
            def _():
                pltpu.async_copy(
                    syn_hbm.at[pl.ds(sg_next, RG), pl.ds(0, C)], buf0, sem0
                )

            return _inner(buf1, k0 + 1, accs)

        accs = lax.fori_loop(0, NCH // 2, _pair, accs)
        for r in range(RG):
            res[g * RG + r] = accs[r]

    pltpu.make_async_copy(
        syn_hbm.at[pl.ds(0, RG), pl.ds(0, C)], buf0, sem0
    ).wait()
    pltpu.sync_copy(res, out_hbm.at[wid])


def _sc_overlaps(basal_synapses, img_flat):
    mesh = plsc.VectorSubcoreMesh(
        core_axis_name="c", subcore_axis_name="s", num_cores=2, num_subcores=16
    )
    return pl.kernel(
        _sc_body,
        out_type=jax.ShapeDtypeStruct((NW, RPW, L), jnp.float32),
        mesh=mesh,
        scratch_types=[
            pltpu.VMEM((COLS,), jnp.float32),
            pltpu.VMEM((RG, C), jnp.float32),
            pltpu.VMEM((RG, C), jnp.float32),
            pltpu.VMEM((RPW, L), jnp.float32),
            pltpu.SemaphoreType.DMA,
            pltpu.SemaphoreType.DMA,
        ],
    )(basal_synapses, img_flat)


def _tc_body(img_ref, syn_ref, out_ref):
    feat = (img_ref[...] > 0.7).astype(jnp.float32)
    out_ref[...] = jnp.sum(syn_ref[...] * feat, axis=1, keepdims=True)


def _tc_matvec(image, basal_synapses):
    return pl.pallas_call(
        _tc_body,
        grid=(TC_ROWS // BLOCK_R,),
        in_specs=[
            pl.BlockSpec((1, COLS), lambda i: (0, 0)),
            pl.BlockSpec((BLOCK_R, COLS), lambda i: (i + TC_START // BLOCK_R, 0)),
        ],
        out_specs=pl.BlockSpec((BLOCK_R, 1), lambda i: (i, 0)),
        out_shape=jax.ShapeDtypeStruct((TC_ROWS, 1), jnp.float32),
    )(image, basal_synapses)


def _fold_body(acc_ref, tc_ref, ov_ref, lbl_ref):
    x = acc_ref[...]
    s = jnp.sum(x, axis=1, keepdims=True)
    ov = jnp.concatenate([s[:TC_START, :], tc_ref[...]], axis=0)
    ov_ref[...] = ov
    idx = lax.broadcasted_iota(jnp.int32, (ROWS, 1), 0)
    m = jnp.max(ov)
    lbl_ref[0] = jnp.min(jnp.where(ov == m, idx, ROWS))


def _fold_argmax(acc, tc_part):
    ov, lbl = pl.pallas_call(
        _fold_body,
        out_specs=[
            pl.BlockSpec(memory_space=pltpu.VMEM),
            pl.BlockSpec(memory_space=pltpu.SMEM),
        ],
        out_shape=[
            jax.ShapeDtypeStruct((ROWS, 1), jnp.float32),
            jax.ShapeDtypeStruct((1,), jnp.int32),
        ],
    )(acc, tc_part)
    return ov.reshape(ROWS), lbl[0]


def _fold_body_sc(acc_ref, ov_ref, lbl_ref):
    x = acc_ref[...]
    s = jnp.sum(x, axis=1, keepdims=True)
    ov_ref[...] = s[:ROWS, :]
    idx = lax.broadcasted_iota(jnp.int32, (SC_ROWS, 1), 0)
    sv = jnp.where(idx < ROWS, s, -1.0)
    m = jnp.max(sv)
    lbl_ref[0] = jnp.min(jnp.where(sv == m, idx, SC_ROWS))


def kernel(image, basal_synapses):
    sc_acc = _sc_overlaps(basal_synapses, image.reshape(COLS))
    ov, lbl = pl.pallas_call(
        _fold_body_sc,
        out_specs=[
            pl.BlockSpec(memory_space=pltpu.VMEM),
            pl.BlockSpec(memory_space=pltpu.SMEM),
        ],
        out_shape=[
            jax.ShapeDtypeStruct((ROWS, 1), jnp.float32),
            jax.ShapeDtypeStruct((1,), jnp.int32),
        ],
    )(sc_acc.reshape(SC_ROWS, L))
    return ov.reshape(ROWS), lbl[0]


# speedup vs baseline: 1.8888x; 1.8888x over previous
import functools

import jax
import jax.numpy as jnp
from jax import lax
from jax.experimental import pallas as pl
from jax.experimental.pallas import tpu as pltpu
from jax.experimental.pallas import tpu_sc as plsc

ROWS = 1000
COLS = 65536
L = 16

NW = 32
RPW = 8
SC_ROWS = NW * RPW
SC_COLS = 32768
RG = 8
NG = RPW // RG
C = 2048
NCH = SC_COLS // C
U = 4

TC_START = 240
BLOCK_R = 40
TC_ROWS = ROWS - TC_START


def _sc_body(syn_hbm, img_hbm, out_hbm, feat, buf0, buf1, res, sem0, sem1):
    wid = lax.axis_index("s") * 2 + lax.axis_index("c")
    base = wid * RPW

    img_cp = pltpu.async_copy(img_hbm, feat, sem1)
    s0 = jnp.minimum(base, ROWS - RG)
    pltpu.async_copy(syn_hbm.at[pl.ds(s0, RG), pl.ds(0, C)], buf0, sem0)
    img_cp.wait()

    ones = jnp.full((L,), 1.0, jnp.float32)
    zeros = jnp.zeros((L,), jnp.float32)

    def _thresh(i, _):
        for u in range(8):
            off = (i * 8 + u) * L
            v = feat[pl.ds(off, L)]
            feat[pl.ds(off, L)] = jnp.where(v > 0.7, ones, zeros)
        return 0

    lax.fori_loop(0, SC_COLS // (L * 8), _thresh, 0)

    def _inner(bufc, k, accs):
        def body(j, accs):
            for u in range(U):
                off = (j * U + u) * L
                fv = feat[pl.ds(k * C + off, L)]
                accs = tuple(
                    accs[r] + bufc[r, pl.ds(off, L)] * fv for r in range(RG)
                )
            return accs

        return lax.fori_loop(0, C // (L * U), body, accs)

    for g in range(NG):
        sg = jnp.minimum(base + g * RG, ROWS - RG)
        sg_next = jnp.minimum(base + (g + 1) * RG, ROWS - RG)
        accs = tuple(jnp.zeros((L,), jnp.float32) for _ in range(RG))

        def _pair(p, accs, sg=sg, sg_next=sg_next):
            k0 = 2 * p
            pltpu.make_async_copy(
                syn_hbm.at[pl.ds(sg, RG), pl.ds(0, C)], buf0, sem0
            ).wait()
            pltpu.async_copy(
                syn_hbm.at[pl.ds(sg, RG), pl.ds((k0 + 1) * C, C)], buf1, sem1
            )
            accs = _inner(buf0, k0, accs)
            pltpu.make_async_copy(
                syn_hbm.at[pl.ds(sg, RG), pl.ds(0, C)], buf1, sem1
            ).wait()

            @pl.when(k0 + 2 < NCH)
            def _():
                pltpu.async_copy(
                    syn_hbm.at[pl.ds(sg, RG), pl.ds((k0 + 2) * C, C)], buf0, sem0
                )

            @pl.when(k0 + 2 >= NCH)
            def _():
                pltpu.async_copy(
                    syn_hbm.at[pl.ds(sg_next, RG), pl.ds(0, C)], buf0, sem0
                )

            return _inner(buf1, k0 + 1, accs)

        accs = lax.fori_loop(0, NCH // 2, _pair, accs)
        for r in range(RG):
            res[g * RG + r] = accs[r]

    pltpu.make_async_copy(
        syn_hbm.at[pl.ds(0, RG), pl.ds(0, C)], buf0, sem0
    ).wait()
    pltpu.sync_copy(res, out_hbm.at[wid])


def _sc_overlaps(basal_synapses, img_flat):
    mesh = plsc.VectorSubcoreMesh(
        core_axis_name="c", subcore_axis_name="s", num_cores=2, num_subcores=16
    )
    return pl.kernel(
        _sc_body,
        out_type=jax.ShapeDtypeStruct((NW, RPW, L), jnp.float32),
        mesh=mesh,
        scratch_types=[
            pltpu.VMEM((SC_COLS,), jnp.float32),
            pltpu.VMEM((RG, C), jnp.float32),
            pltpu.VMEM((RG, C), jnp.float32),
            pltpu.VMEM((RPW, L), jnp.float32),
            pltpu.SemaphoreType.DMA,
            pltpu.SemaphoreType.DMA,
        ],
    )(basal_synapses, img_flat)


def _tc_body(img_ref, syn_ref, out_ref):
    feat = (img_ref[...] > 0.7).astype(jnp.float32)
    out_ref[...] = jnp.sum(syn_ref[...] * feat, axis=1, keepdims=True)


def _tc_matvec(image, basal_synapses):
    return pl.pallas_call(
        _tc_body,
        grid=(TC_ROWS // BLOCK_R,),
        in_specs=[
            pl.BlockSpec((1, COLS), lambda i: (0, 0)),
            pl.BlockSpec((BLOCK_R, COLS), lambda i: (i + TC_START // BLOCK_R, 0)),
        ],
        out_specs=pl.BlockSpec((BLOCK_R, 1), lambda i: (i, 0)),
        out_shape=jax.ShapeDtypeStruct((TC_ROWS, 1), jnp.float32),
    )(image, basal_synapses)


def _fold_body(acc_ref, tc2_ref, tc_ref, ov_ref, lbl_ref):
    x = acc_ref[...]
    s = jnp.sum(x, axis=1, keepdims=True) + tc2_ref[...]
    ov = jnp.concatenate([s[:TC_START, :], tc_ref[...]], axis=0)
    ov_ref[...] = ov
    idx = lax.broadcasted_iota(jnp.int32, (ROWS, 1), 0)
    m = jnp.max(ov)
    lbl_ref[0] = jnp.min(jnp.where(ov == m, idx, ROWS))


def _fold_argmax(acc, tc2_part, tc_part):
    ov, lbl = pl.pallas_call(
        _fold_body,
        out_specs=[
            pl.BlockSpec(memory_space=pltpu.VMEM),
            pl.BlockSpec(memory_space=pltpu.SMEM),
        ],
        out_shape=[
            jax.ShapeDtypeStruct((ROWS, 1), jnp.float32),
            jax.ShapeDtypeStruct((1,), jnp.int32),
        ],
    )(acc, tc2_part, tc_part)
    return ov.reshape(ROWS), lbl[0]


def _tc2_body(img_ref, syn_ref, out_ref):
    feat = (img_ref[...] > 0.7).astype(jnp.float32)
    out_ref[...] = jnp.sum(syn_ref[...] * feat, axis=1, keepdims=True)


def _tc_cols2(image, basal_synapses):
    blk = 32
    return pl.pallas_call(
        _tc2_body,
        grid=(SC_ROWS // blk,),
        in_specs=[
            pl.BlockSpec((1, COLS - SC_COLS), lambda i: (0, 1)),
            pl.BlockSpec((blk, COLS - SC_COLS), lambda i: (i, 1)),
        ],
        out_specs=pl.BlockSpec((blk, 1), lambda i: (i, 0)),
        out_shape=jax.ShapeDtypeStruct((SC_ROWS, 1), jnp.float32),
    )(image, basal_synapses)


def kernel(image, basal_synapses):
    sc_acc = _sc_overlaps(basal_synapses, image.reshape(COLS)[:SC_COLS])
    tc2_part = _tc_cols2(image, basal_synapses)
    tc_part = _tc_matvec(image, basal_synapses)
    return _fold_argmax(sc_acc.reshape(SC_ROWS, L), tc2_part, tc_part)


# speedup vs baseline: 1.9649x; 1.0403x over previous
import functools

import jax
import jax.numpy as jnp
from jax import lax
from jax.experimental import pallas as pl
from jax.experimental.pallas import tpu as pltpu
from jax.experimental.pallas import tpu_sc as plsc

ROWS = 1000
COLS = 65536
L = 16

NW = 32
RPW = 8
SC_ROWS = NW * RPW
SC_COLS = 20480
SC_COL_OFF = COLS - SC_COLS
TC2_ROWS = 240
TC2_BLK = 48
RG = 8
NG = RPW // RG
C = 2048
NCH = SC_COLS // C
U = 4

TC_START = 240
BLOCK_R = 40
TC_ROWS = ROWS - TC_START


def _sc_body(syn_hbm, img_hbm, out_hbm, feat, buf0, buf1, res, sem0, sem1):
    wid = lax.axis_index("s") * 2 + lax.axis_index("c")
    base = wid * RPW

    img_cp = pltpu.async_copy(img_hbm, feat, sem1)
    s0 = jnp.minimum(base, ROWS - RG)
    pltpu.async_copy(syn_hbm.at[pl.ds(s0, RG), pl.ds(SC_COL_OFF, C)], buf0, sem0)
    img_cp.wait()

    ones = jnp.full((L,), 1.0, jnp.float32)
    zeros = jnp.zeros((L,), jnp.float32)

    def _thresh(i, _):
        for u in range(8):
            off = (i * 8 + u) * L
            v = feat[pl.ds(off, L)]
            feat[pl.ds(off, L)] = jnp.where(v > 0.7, ones, zeros)
        return 0

    lax.fori_loop(0, SC_COLS // (L * 8), _thresh, 0)

    def _inner(bufc, k, accs):
        def body(j, accs):
            for u in range(U):
                off = (j * U + u) * L
                fv = feat[pl.ds(k * C + off, L)]
                accs = tuple(
                    accs[r] + bufc[r, pl.ds(off, L)] * fv for r in range(RG)
                )
            return accs

        return lax.fori_loop(0, C // (L * U), body, accs)

    for g in range(NG):
        sg = jnp.minimum(base + g * RG, ROWS - RG)
        sg_next = jnp.minimum(base + (g + 1) * RG, ROWS - RG)
        accs = tuple(jnp.zeros((L,), jnp.float32) for _ in range(RG))

        def _pair(p, accs, sg=sg, sg_next=sg_next):
            k0 = 2 * p
            pltpu.make_async_copy(
                syn_hbm.at[pl.ds(sg, RG), pl.ds(0, C)], buf0, sem0
            ).wait()
            pltpu.async_copy(
                syn_hbm.at[pl.ds(sg, RG), pl.ds(SC_COL_OFF + (k0 + 1) * C, C)],
                buf1, sem1,
            )
            accs = _inner(buf0, k0, accs)
            pltpu.make_async_copy(
                syn_hbm.at[pl.ds(sg, RG), pl.ds(0, C)], buf1, sem1
            ).wait()

            @pl.when(k0 + 2 < NCH)
            def _():
                pltpu.async_copy(
                    syn_hbm.at[pl.ds(sg, RG), pl.ds(SC_COL_OFF + (k0 + 2) * C, C)],
                    buf0, sem0,
                )

            @pl.when(k0 + 2 >= NCH)
            def _():
                pltpu.async_copy(
                    syn_hbm.at[pl.ds(sg_next, RG), pl.ds(SC_COL_OFF, C)], buf0, sem0
                )

            return _inner(buf1, k0 + 1, accs)

        accs = lax.fori_loop(0, NCH // 2, _pair, accs)
        for r in range(RG):
            res[g * RG + r] = accs[r]

    pltpu.make_async_copy(
        syn_hbm.at[pl.ds(0, RG), pl.ds(0, C)], buf0, sem0
    ).wait()
    pltpu.sync_copy(res, out_hbm.at[wid])


def _sc_overlaps(basal_synapses, img_flat):
    mesh = plsc.VectorSubcoreMesh(
        core_axis_name="c", subcore_axis_name="s", num_cores=2, num_subcores=16
    )
    return pl.kernel(
        _sc_body,
        out_type=jax.ShapeDtypeStruct((NW, RPW, L), jnp.float32),
        mesh=mesh,
        scratch_types=[
            pltpu.VMEM((SC_COLS,), jnp.float32),
            pltpu.VMEM((RG, C), jnp.float32),
            pltpu.VMEM((RG, C), jnp.float32),
            pltpu.VMEM((RPW, L), jnp.float32),
            pltpu.SemaphoreType.DMA,
            pltpu.SemaphoreType.DMA,
        ],
    )(basal_synapses, img_flat)


def _tc_body(img_ref, syn_ref, out_ref):
    feat = (img_ref[...] > 0.7).astype(jnp.float32)
    out_ref[...] = jnp.sum(syn_ref[...] * feat, axis=1, keepdims=True)


def _tc_matvec(image, basal_synapses):
    return pl.pallas_call(
        _tc_body,
        grid=(TC_ROWS // BLOCK_R,),
        in_specs=[
            pl.BlockSpec((1, COLS), lambda i: (0, 0)),
            pl.BlockSpec((BLOCK_R, COLS), lambda i: (i + TC_START // BLOCK_R, 0)),
        ],
        out_specs=pl.BlockSpec((BLOCK_R, 1), lambda i: (i, 0)),
        out_shape=jax.ShapeDtypeStruct((TC_ROWS, 1), jnp.float32),
    )(image, basal_synapses)


def _fold_body(acc_ref, tc2_ref, tc_ref, ov_ref, lbl_ref):
    x = acc_ref[...]
    s = jnp.sum(x, axis=1, keepdims=True)[:TC2_ROWS, :] + tc2_ref[...]
    ov = jnp.concatenate([s, tc_ref[...]], axis=0)
    ov_ref[...] = ov
    idx = lax.broadcasted_iota(jnp.int32, (ROWS, 1), 0)
    m = jnp.max(ov)
    lbl_ref[0] = jnp.min(jnp.where(ov == m, idx, ROWS))


def _fold_argmax(acc, tc2_part, tc_part):
    ov, lbl = pl.pallas_call(
        _fold_body,
        out_specs=[
            pl.BlockSpec(memory_space=pltpu.VMEM),
            pl.BlockSpec(memory_space=pltpu.SMEM),
        ],
        out_shape=[
            jax.ShapeDtypeStruct((ROWS, 1), jnp.float32),
            jax.ShapeDtypeStruct((1,), jnp.int32),
        ],
    )(acc, tc2_part, tc_part)
    return ov.reshape(ROWS), lbl[0]


def _tc2_body(img_ref, syn_ref, out_ref):
    feat = (img_ref[...] > 0.7).astype(jnp.float32)
    out_ref[...] = jnp.sum(syn_ref[...] * feat, axis=1, keepdims=True)


def _tc_cols2(image, basal_synapses):
    return pl.pallas_call(
        _tc2_body,
        grid=(TC2_ROWS // TC2_BLK,),
        in_specs=[
            pl.BlockSpec((1, SC_COL_OFF), lambda i: (0, 0)),
            pl.BlockSpec((TC2_BLK, SC_COL_OFF), lambda i: (i, 0)),
        ],
        out_specs=pl.BlockSpec((TC2_BLK, 1), lambda i: (i, 0)),
        out_shape=jax.ShapeDtypeStruct((TC2_ROWS, 1), jnp.float32),
    )(image, basal_synapses)


def kernel(image, basal_synapses):
    sc_acc = _sc_overlaps(basal_synapses, image.reshape(COLS)[SC_COL_OFF:])
    tc2_part = _tc_cols2(image, basal_synapses)
    tc_part = _tc_matvec(image, basal_synapses)
    return _fold_argmax(sc_acc.reshape(SC_ROWS, L), tc2_part, tc_part)


# speedup vs baseline: 2.5521x; 1.2988x over previous
import jax
import jax.numpy as jnp
from jax.experimental import pallas as pl
from jax.experimental.pallas import tpu as pltpu

ROWS = 1000
COLS = 65536
BLOCK_R = 40


def _body(img_ref, syn_ref, out_ref, lbl_ref, best_ref):
    i = pl.program_id(0)

    @pl.when(i == 0)
    def _init():
        best_ref[0] = -1.0
        lbl_ref[0] = 0

    feat = (img_ref[...] > 0.7).astype(jnp.float32)
    partial = jnp.sum(syn_ref[...] * feat, axis=1, keepdims=True)
    out_ref[...] = partial

    bmax = jnp.max(partial)
    idx2d = jax.lax.broadcasted_iota(jnp.int32, (BLOCK_R, 1), 0)
    local_arg = jnp.min(jnp.where(partial == bmax, idx2d, BLOCK_R))
    gidx = i * BLOCK_R + local_arg
    pred = bmax > best_ref[0]
    best_ref[0] = jnp.where(pred, bmax, best_ref[0])
    lbl_ref[0] = jnp.where(pred, gidx, lbl_ref[0])


def kernel(image, basal_synapses):
    overlaps2d, lbl = pl.pallas_call(
        _body,
        grid=(ROWS // BLOCK_R,),
        in_specs=[
            pl.BlockSpec((1, COLS), lambda i: (0, 0)),
            pl.BlockSpec((BLOCK_R, COLS), lambda i: (i, 0)),
        ],
        out_specs=[
            pl.BlockSpec((BLOCK_R, 1), lambda i: (i, 0)),
            pl.BlockSpec(memory_space=pltpu.SMEM),
        ],
        out_shape=[
            jax.ShapeDtypeStruct((ROWS, 1), jnp.float32),
            jax.ShapeDtypeStruct((1,), jnp.int32),
        ],
        scratch_shapes=[pltpu.SMEM((1,), jnp.float32)],
    )(image, basal_synapses)
    return overlaps2d.reshape(ROWS), lbl[0]
